# baseline (speedup 1.0000x reference)
import functools

import jax
import jax.numpy as jnp
import numpy as np
from jax import lax
from jax.experimental import pallas as pl
from jax.experimental.pallas import tpu as pltpu
from jax.experimental.pallas import tpu_sc as plsc

N = 524288
N_LEVELS = 16
F = 2
LOG2_T = 19
TABLE_SIZE = 1 << LOG2_T
BASE_RES = 16
SCALE = 1.38
WIDTH = 64
OUT_DIM = 16
IN_DIM = N_LEVELS * F

NC = 2
NS = 16
L = 16
NW = NC * NS
PW = N // NW
B = 64
CH = PW // B
RPP = N_LEVELS * 8
R = B * RPP

P1 = 2654435761 - (1 << 32)
P2 = 805459861
MASK = TABLE_SIZE - 1

RES = [float(np.floor(BASE_RES * SCALE**l)) for l in range(N_LEVELS)]

_mesh = plsc.VectorSubcoreMesh(core_axis_name="c", subcore_axis_name="s")


@functools.partial(
    pl.kernel,
    out_type=jax.ShapeDtypeStruct((IN_DIM, N), jnp.float32),
    mesh=_mesh,
    compiler_params=pltpu.CompilerParams(
        use_tc_tiling_on_sc=False, needs_layout_passes=False
    ),
    scratch_types=[
        pltpu.VMEM((B, 3), jnp.float32),
        pltpu.VMEM((R,), jnp.int32),
        pltpu.VMEM((R, F), jnp.float32),
        pltpu.VMEM((IN_DIM * B,), jnp.float32),
        pltpu.SemaphoreType.DMA,
    ],
)
def _encode(xyz, tab, enc, xyzv, idxv, rowsv, outv, sem):
    wid = lax.axis_index("s") * NC + lax.axis_index("c")
    iota = lax.iota(jnp.int32, L)
    d0 = jnp.zeros((L,), jnp.int32)
    d1 = jnp.full((L,), 1, jnp.int32)
    d2 = jnp.full((L,), 2, jnp.int32)

    def chunk_body(ch, carry):
        base = wid * PW + ch * B
        pltpu.sync_copy(xyz.at[pl.ds(base, B)], xyzv)

        def phase_a(v, c2):
            pids = v * L + iota
            x = plsc.load_gather(xyzv, [pids, d0])
            y = plsc.load_gather(xyzv, [pids, d1])
            z = plsc.load_gather(xyzv, [pids, d2])
            for l in range(N_LEVELS):
                res = RES[l]
                px = (x * res).astype(jnp.int32)
                py = (y * res).astype(jnp.int32)
                pz = (z * res).astype(jnp.int32)
                loff = l * TABLE_SIZE
                hyt = py * P1
                hy = (jnp.bitwise_and(hyt, MASK), jnp.bitwise_and(hyt + P1, MASK))
                hzt = pz * P2
                hz = (
                    jnp.bitwise_or(jnp.bitwise_and(hzt, MASK), loff),
                    jnp.bitwise_or(jnp.bitwise_and(hzt + P2, MASK), loff),
                )
                hxy = (px ^ hy[0], px ^ hy[1], (px + 1) ^ hy[0], (px + 1) ^ hy[1])
                for c in range(8):
                    bx, by, bz = (c >> 2) & 1, (c >> 1) & 1, c & 1
                    h = hxy[bx * 2 + by] ^ hz[bz]
                    idxv[pl.ds((l * 8 + c) * B + v * L, L)] = h
            return c2

        lax.fori_loop(0, B // L, phase_a, 0)

        pltpu.async_copy(tab.at[idxv], rowsv, sem).wait()

        def phase_b(v, c2):
            pids = v * L + iota
            x = plsc.load_gather(xyzv, [pids, d0])
            y = plsc.load_gather(xyzv, [pids, d1])
            z = plsc.load_gather(xyzv, [pids, d2])
            zero = d0
            one = d1
            for l in range(N_LEVELS):
                res = RES[l]
                posx, posy, posz = x * res, y * res, z * res
                px = posx.astype(jnp.int32)
                py = posy.astype(jnp.int32)
                pz = posz.astype(jnp.int32)
                fx = posx - px.astype(jnp.float32)
                fy = posy - py.astype(jnp.float32)
                fz = posz - pz.astype(jnp.float32)
                wx = (1.0 - fx, fx)
                wy = (1.0 - fy, fy)
                wz = (1.0 - fz, fz)
                wyz = (wy[0] * wz[0], wy[0] * wz[1], wy[1] * wz[0], wy[1] * wz[1])
                acc0 = jnp.zeros((L,), jnp.float32)
                acc1 = jnp.zeros((L,), jnp.float32)
                for c in range(8):
                    bx, by, bz = (c >> 2) & 1, (c >> 1) & 1, c & 1
                    w = wx[bx] * wyz[by * 2 + bz]
                    rows = (l * 8 + c) * B + pids
                    f0 = plsc.load_gather(rowsv, [rows, zero])
                    f1 = plsc.load_gather(rowsv, [rows, one])
                    acc0 = acc0 + f0 * w
                    acc1 = acc1 + f1 * w
                outv[pl.ds((2 * l) * B + v * L, L)] = acc0
                outv[pl.ds((2 * l + 1) * B + v * L, L)] = acc1
            return c2

        lax.fori_loop(0, B // L, phase_b, 0)

        for f in range(IN_DIM):
            pltpu.sync_copy(
                outv.at[pl.ds(f * B, B)], enc.at[f, pl.ds(base, B)]
            )
        return carry

    lax.fori_loop(0, CH, chunk_body, 0)


BN = 4096


def _mlp_body(encT_ref, w0, b0, w1, b1, w2, b2, out_ref):
    dn = (((0,), (0,)), ((), ()))
    h = jnp.maximum(
        lax.dot_general(encT_ref[...], w0[...], dn, preferred_element_type=jnp.float32)
        + b0[...],
        0.0,
    )
    h = jnp.maximum(
        jnp.dot(h, w1[...], preferred_element_type=jnp.float32) + b1[...], 0.0
    )
    out_ref[...] = jnp.dot(h, w2[...], preferred_element_type=jnp.float32) + b2[...]


def _mlp(encT, W0, b0, W1, b1, W2, b2):
    return pl.pallas_call(
        _mlp_body,
        grid=(N // BN,),
        in_specs=[
            pl.BlockSpec((IN_DIM, BN), lambda i: (0, i)),
            pl.BlockSpec((IN_DIM, WIDTH), lambda i: (0, 0)),
            pl.BlockSpec((1, WIDTH), lambda i: (0, 0)),
            pl.BlockSpec((WIDTH, WIDTH), lambda i: (0, 0)),
            pl.BlockSpec((1, WIDTH), lambda i: (0, 0)),
            pl.BlockSpec((WIDTH, OUT_DIM), lambda i: (0, 0)),
            pl.BlockSpec((1, OUT_DIM), lambda i: (0, 0)),
        ],
        out_specs=pl.BlockSpec((BN, OUT_DIM), lambda i: (i, 0)),
        out_shape=jax.ShapeDtypeStruct((N, OUT_DIM), jnp.float32),
    )(
        encT,
        W0,
        b0.reshape(1, WIDTH),
        W1,
        b1.reshape(1, WIDTH),
        W2,
        b2.reshape(1, OUT_DIM),
    )


def kernel(xyz, tables, W0, b0, W1, b1, W2, b2):
    tab = tables.reshape(N_LEVELS * TABLE_SIZE, F)
    encT = _encode(xyz, tab)
    return _mlp(encT, W0, b0, W1, b1, W2, b2)


# speedup vs baseline: 2.9226x; 2.9226x over previous
import functools

import jax
import jax.numpy as jnp
import numpy as np
from jax import lax
from jax.experimental import pallas as pl
from jax.experimental.pallas import tpu as pltpu
from jax.experimental.pallas import tpu_sc as plsc

N = 524288
N_LEVELS = 16
F = 2
LOG2_T = 19
TABLE_SIZE = 1 << LOG2_T
BASE_RES = 16
SCALE = 1.38
WIDTH = 64
OUT_DIM = 16
IN_DIM = N_LEVELS * F

NC = 2
NS = 16
L = 16
NW = NC * NS
PW = N // NW
B = 256
CH = PW // B
RPP = N_LEVELS * 8
R = B * RPP

P1 = 2654435761 - (1 << 32)
P2 = 805459861
MASK = TABLE_SIZE - 1

RES = [float(np.floor(BASE_RES * SCALE**l)) for l in range(N_LEVELS)]

_mesh = plsc.VectorSubcoreMesh(core_axis_name="c", subcore_axis_name="s")


@functools.partial(
    pl.kernel,
    out_type=jax.ShapeDtypeStruct((IN_DIM * N,), jnp.float32),
    mesh=_mesh,
    compiler_params=pltpu.CompilerParams(
        use_tc_tiling_on_sc=False, needs_layout_passes=False
    ),
    scratch_types=[
        pltpu.VMEM((B * 3,), jnp.float32),
        pltpu.VMEM((R,), jnp.int32),
        pltpu.VMEM((R,), jnp.float32),
        pltpu.VMEM((R,), jnp.float32),
        pltpu.VMEM((IN_DIM * B,), jnp.float32),
        pltpu.SemaphoreType.DMA,
        pltpu.SemaphoreType.DMA,
    ],
)
def _encode(xyz, tab0, tab1, enc, xyzv, idxv, rows0, rows1, outv, sem0, sem1):
    wid = lax.axis_index("s") * NC + lax.axis_index("c")
    iota = lax.iota(jnp.int32, L)

    def chunk_body(ch, carry):
        base = wid * PW + ch * B
        pltpu.sync_copy(xyz.at[pl.ds(base * 3, B * 3)], xyzv)

        def phase_a(v, c2):
            pids3 = (v * L + iota) * 3
            x = plsc.load_gather(xyzv, [pids3])
            y = plsc.load_gather(xyzv, [pids3 + 1])
            z = plsc.load_gather(xyzv, [pids3 + 2])
            for l in range(N_LEVELS):
                res = RES[l]
                px = (x * res).astype(jnp.int32)
                py = (y * res).astype(jnp.int32)
                pz = (z * res).astype(jnp.int32)
                loff = l * TABLE_SIZE
                hyt = py * P1
                hy = (jnp.bitwise_and(hyt, MASK), jnp.bitwise_and(hyt + P1, MASK))
                hzt = pz * P2
                hz = (
                    jnp.bitwise_or(jnp.bitwise_and(hzt, MASK), loff),
                    jnp.bitwise_or(jnp.bitwise_and(hzt + P2, MASK), loff),
                )
                hxy = (px ^ hy[0], px ^ hy[1], (px + 1) ^ hy[0], (px + 1) ^ hy[1])
                for c in range(8):
                    bx, by, bz = (c >> 2) & 1, (c >> 1) & 1, c & 1
                    h = hxy[bx * 2 + by] ^ hz[bz]
                    idxv[pl.ds((l * 8 + c) * B + v * L, L)] = h
            return c2

        lax.fori_loop(0, B // L, phase_a, 0)

        cp0 = pltpu.async_copy(tab0.at[idxv], rows0, sem0)
        cp1 = pltpu.async_copy(tab1.at[idxv], rows1, sem1)
        cp0.wait()
        cp1.wait()

        def phase_b(v, c2):
            pids = v * L + iota
            pids3 = pids * 3
            x = plsc.load_gather(xyzv, [pids3])
            y = plsc.load_gather(xyzv, [pids3 + 1])
            z = plsc.load_gather(xyzv, [pids3 + 2])
            for l in range(N_LEVELS):
                res = RES[l]
                posx, posy, posz = x * res, y * res, z * res
                px = posx.astype(jnp.int32)
                py = posy.astype(jnp.int32)
                pz = posz.astype(jnp.int32)
                fx = posx - px.astype(jnp.float32)
                fy = posy - py.astype(jnp.float32)
                fz = posz - pz.astype(jnp.float32)
                wx = (1.0 - fx, fx)
                wy = (1.0 - fy, fy)
                wz = (1.0 - fz, fz)
                wyz = (wy[0] * wz[0], wy[0] * wz[1], wy[1] * wz[0], wy[1] * wz[1])
                acc0 = jnp.zeros((L,), jnp.float32)
                acc1 = jnp.zeros((L,), jnp.float32)
                for c in range(8):
                    bx, by, bz = (c >> 2) & 1, (c >> 1) & 1, c & 1
                    w = wx[bx] * wyz[by * 2 + bz]
                    rows = (l * 8 + c) * B + pids
                    f0 = plsc.load_gather(rows0, [rows])
                    f1 = plsc.load_gather(rows1, [rows])
                    acc0 = acc0 + f0 * w
                    acc1 = acc1 + f1 * w
                outv[pl.ds((2 * l) * B + v * L, L)] = acc0
                outv[pl.ds((2 * l + 1) * B + v * L, L)] = acc1
            return c2

        lax.fori_loop(0, B // L, phase_b, 0)

        for f in range(IN_DIM):
            pltpu.sync_copy(
                outv.at[pl.ds(f * B, B)], enc.at[pl.ds(f * N + base, B)]
            )
        return carry

    lax.fori_loop(0, CH, chunk_body, 0)


BN = 4096


def _mlp_body(encT_ref, w0, b0, w1, b1, w2, b2, out_ref):
    dn = (((0,), (0,)), ((), ()))
    h = jnp.maximum(
        lax.dot_general(encT_ref[...], w0[...], dn, preferred_element_type=jnp.float32)
        + b0[...],
        0.0,
    )
    h = jnp.maximum(
        jnp.dot(h, w1[...], preferred_element_type=jnp.float32) + b1[...], 0.0
    )
    out_ref[...] = jnp.dot(h, w2[...], preferred_element_type=jnp.float32) + b2[...]


def _mlp(encT, W0, b0, W1, b1, W2, b2):
    return pl.pallas_call(
        _mlp_body,
        grid=(N // BN,),
        in_specs=[
            pl.BlockSpec((IN_DIM, BN), lambda i: (0, i)),
            pl.BlockSpec((IN_DIM, WIDTH), lambda i: (0, 0)),
            pl.BlockSpec((1, WIDTH), lambda i: (0, 0)),
            pl.BlockSpec((WIDTH, WIDTH), lambda i: (0, 0)),
            pl.BlockSpec((1, WIDTH), lambda i: (0, 0)),
            pl.BlockSpec((WIDTH, OUT_DIM), lambda i: (0, 0)),
            pl.BlockSpec((1, OUT_DIM), lambda i: (0, 0)),
        ],
        out_specs=pl.BlockSpec((BN, OUT_DIM), lambda i: (i, 0)),
        out_shape=jax.ShapeDtypeStruct((N, OUT_DIM), jnp.float32),
    )(
        encT,
        W0,
        b0.reshape(1, WIDTH),
        W1,
        b1.reshape(1, WIDTH),
        W2,
        b2.reshape(1, OUT_DIM),
    )


def kernel(xyz, tables, W0, b0, W1, b1, W2, b2):
    tab0 = tables[:, :, 0].reshape(N_LEVELS * TABLE_SIZE)
    tab1 = tables[:, :, 1].reshape(N_LEVELS * TABLE_SIZE)
    enc1 = _encode(xyz.reshape(N * 3), tab0, tab1)
    return _mlp(enc1.reshape(IN_DIM, N), W0, b0, W1, b1, W2, b2)


# speedup vs baseline: 2.9437x; 1.0072x over previous
import functools

import jax
import jax.numpy as jnp
import numpy as np
from jax import lax
from jax.experimental import pallas as pl
from jax.experimental.pallas import tpu as pltpu
from jax.experimental.pallas import tpu_sc as plsc

N = 524288
N_LEVELS = 16
F = 2
LOG2_T = 19
TABLE_SIZE = 1 << LOG2_T
BASE_RES = 16
SCALE = 1.38
WIDTH = 64
OUT_DIM = 16
IN_DIM = N_LEVELS * F

NC = 2
NS = 16
L = 16
NW = NC * NS
PW = N // NW
B = 128
CH = PW // B
RPP = N_LEVELS * 8
R = B * RPP

P1 = 2654435761 - (1 << 32)
P2 = 805459861
MASK = TABLE_SIZE - 1

RES = [float(np.floor(BASE_RES * SCALE**l)) for l in range(N_LEVELS)]

_mesh = plsc.VectorSubcoreMesh(core_axis_name="c", subcore_axis_name="s")


@functools.partial(
    pl.kernel,
    out_type=jax.ShapeDtypeStruct((IN_DIM, N), jnp.float32),
    mesh=_mesh,
    compiler_params=pltpu.CompilerParams(
        use_tc_tiling_on_sc=False, needs_layout_passes=False
    ),
    scratch_types=[
        pltpu.VMEM((B * 3,), jnp.float32),
        pltpu.VMEM((B * 3,), jnp.float32),
        pltpu.VMEM((R,), jnp.int32),
        pltpu.VMEM((R,), jnp.int32),
        pltpu.VMEM((R,), jnp.float32),
        pltpu.VMEM((R,), jnp.float32),
        pltpu.VMEM((R,), jnp.float32),
        pltpu.VMEM((R,), jnp.float32),
        pltpu.VMEM((IN_DIM, B), jnp.float32),
        pltpu.SemaphoreType.DMA,
        pltpu.SemaphoreType.DMA,
        pltpu.SemaphoreType.DMA,
        pltpu.SemaphoreType.DMA,
    ],
)
def _encode(
    xyz, tab0, tab1, enc,
    xyzv0, xyzv1, idx0, idx1, r0e, r0o, r1e, r1o, outv,
    s0e, s0o, s1e, s1o,
):
    wid = lax.axis_index("s") * NC + lax.axis_index("c")
    iota = lax.iota(jnp.int32, L)

    def stage_and_index(i, xyzv, idxv):
        base = wid * PW + i * B
        pltpu.sync_copy(xyz.at[pl.ds(base * 3, B * 3)], xyzv)

        def phase_a(v, c2):
            pids3 = (v * L + iota) * 3
            x = plsc.load_gather(xyzv, [pids3])
            y = plsc.load_gather(xyzv, [pids3 + 1])
            z = plsc.load_gather(xyzv, [pids3 + 2])
            for l in range(N_LEVELS):
                res = RES[l]
                px = (x * res).astype(jnp.int32)
                py = (y * res).astype(jnp.int32)
                pz = (z * res).astype(jnp.int32)
                loff = l * TABLE_SIZE
                hyt = py * P1
                hy = (jnp.bitwise_and(hyt, MASK), jnp.bitwise_and(hyt + P1, MASK))
                hzt = pz * P2
                hz = (
                    jnp.bitwise_or(jnp.bitwise_and(hzt, MASK), loff),
                    jnp.bitwise_or(jnp.bitwise_and(hzt + P2, MASK), loff),
                )
                hxy = (px ^ hy[0], px ^ hy[1], (px + 1) ^ hy[0], (px + 1) ^ hy[1])
                for c in range(8):
                    bx, by, bz = (c >> 2) & 1, (c >> 1) & 1, c & 1
                    h = hxy[bx * 2 + by] ^ hz[bz]
                    idxv[pl.ds((l * 8 + c) * B + v * L, L)] = h
            return c2

        lax.fori_loop(0, B // L, phase_a, 0)

    def fire(idxv, rA, rB, semA, semB):
        pltpu.async_copy(tab0.at[idxv], rA, semA)
        pltpu.async_copy(tab1.at[idxv], rB, semB)

    def drain(idxv, rA, rB, semA, semB):
        pltpu.make_async_copy(tab0.at[idxv], rA, semA).wait()
        pltpu.make_async_copy(tab1.at[idxv], rB, semB).wait()

    def interp_and_out(i, xyzv, rA, rB):
        base = wid * PW + i * B

        def phase_b(v, c2):
            pids3 = (v * L + iota) * 3
            x = plsc.load_gather(xyzv, [pids3])
            y = plsc.load_gather(xyzv, [pids3 + 1])
            z = plsc.load_gather(xyzv, [pids3 + 2])
            for l in range(N_LEVELS):
                res = RES[l]
                posx, posy, posz = x * res, y * res, z * res
                px = posx.astype(jnp.int32)
                py = posy.astype(jnp.int32)
                pz = posz.astype(jnp.int32)
                fx = posx - px.astype(jnp.float32)
                fy = posy - py.astype(jnp.float32)
                fz = posz - pz.astype(jnp.float32)
                wx = (1.0 - fx, fx)
                wy = (1.0 - fy, fy)
                wz = (1.0 - fz, fz)
                wyz = (wy[0] * wz[0], wy[0] * wz[1], wy[1] * wz[0], wy[1] * wz[1])
                acc0 = jnp.zeros((L,), jnp.float32)
                acc1 = jnp.zeros((L,), jnp.float32)
                for c in range(8):
                    bx, by, bz = (c >> 2) & 1, (c >> 1) & 1, c & 1
                    w = wx[bx] * wyz[by * 2 + bz]
                    s = (l * 8 + c) * B + v * L
                    acc0 = acc0 + rA[pl.ds(s, L)] * w
                    acc1 = acc1 + rB[pl.ds(s, L)] * w
                outv[2 * l, pl.ds(v * L, L)] = acc0
                outv[2 * l + 1, pl.ds(v * L, L)] = acc1
            return c2

        lax.fori_loop(0, B // L, phase_b, 0)
        pltpu.sync_copy(outv, enc.at[:, pl.ds(base, B)])

    stage_and_index(0, xyzv0, idx0)
    fire(idx0, r0e, r1e, s0e, s1e)

    def g_body(g, carry):
        i0 = 2 * g
        i1 = i0 + 1
        stage_and_index(i1, xyzv1, idx1)
        fire(idx1, r0o, r1o, s0o, s1o)
        drain(idx0, r0e, r1e, s0e, s1e)
        interp_and_out(i0, xyzv0, r0e, r1e)

        @pl.when(i1 + 1 < CH)
        def _():
            stage_and_index(i0 + 2, xyzv0, idx0)
            fire(idx0, r0e, r1e, s0e, s1e)

        drain(idx1, r0o, r1o, s0o, s1o)
        interp_and_out(i1, xyzv1, r0o, r1o)
        return carry

    lax.fori_loop(0, CH // 2, g_body, 0)


BN = 4096


def _mlp_body(encT_ref, w0, b0, w1, b1, w2, b2, out_ref):
    dn = (((0,), (0,)), ((), ()))
    h = jnp.maximum(
        lax.dot_general(encT_ref[...], w0[...], dn, preferred_element_type=jnp.float32)
        + b0[...],
        0.0,
    )
    h = jnp.maximum(
        jnp.dot(h, w1[...], preferred_element_type=jnp.float32) + b1[...], 0.0
    )
    out_ref[...] = jnp.dot(h, w2[...], preferred_element_type=jnp.float32) + b2[...]


def _mlp(encT, W0, b0, W1, b1, W2, b2):
    return pl.pallas_call(
        _mlp_body,
        grid=(N // BN,),
        in_specs=[
            pl.BlockSpec((IN_DIM, BN), lambda i: (0, i)),
            pl.BlockSpec((IN_DIM, WIDTH), lambda i: (0, 0)),
            pl.BlockSpec((1, WIDTH), lambda i: (0, 0)),
            pl.BlockSpec((WIDTH, WIDTH), lambda i: (0, 0)),
            pl.BlockSpec((1, WIDTH), lambda i: (0, 0)),
            pl.BlockSpec((WIDTH, OUT_DIM), lambda i: (0, 0)),
            pl.BlockSpec((1, OUT_DIM), lambda i: (0, 0)),
        ],
        out_specs=pl.BlockSpec((BN, OUT_DIM), lambda i: (i, 0)),
        out_shape=jax.ShapeDtypeStruct((N, OUT_DIM), jnp.float32),
    )(
        encT,
        W0,
        b0.reshape(1, WIDTH),
        W1,
        b1.reshape(1, WIDTH),
        W2,
        b2.reshape(1, OUT_DIM),
    )


def kernel(xyz, tables, W0, b0, W1, b1, W2, b2):
    tab0 = tables[:, :, 0].reshape(N_LEVELS * TABLE_SIZE)
    tab1 = tables[:, :, 1].reshape(N_LEVELS * TABLE_SIZE)
    encT = _encode(xyz.reshape(N * 3), tab0, tab1)
    return _mlp(encT, W0, b0, W1, b1, W2, b2)
